# Initial kernel scaffold; baseline (speedup 1.0000x reference)
import jax
import jax.numpy as jnp
from jax.experimental import pallas as pl


def kernel(x, word_table, pos_table):
    raise NotImplementedError("write your pallas kernel here")


# speedup vs baseline: 3.4553x; 3.4553x over previous
import functools

import jax
import jax.numpy as jnp
from jax import lax
from jax.experimental import pallas as pl
from jax.experimental.pallas import tpu as pltpu
from jax.experimental.pallas import tpu_sc as plsc

VOCAB = 100000
MAX_LEN = 200
EMB_DIM = 64
BATCH = 4096
SEQ_LEN = 200

NC, NS = 2, 16
NW = NC * NS
TOTAL_ROWS = BATCH * SEQ_LEN
ROWS_PER_W = TOTAL_ROWS // NW
SEQ_PER_CHUNK = 2
CHUNK = SEQ_PER_CHUNK * SEQ_LEN
N_CHUNKS = ROWS_PER_W // CHUNK
SUBG = 80
NSUB = CHUNK // SUBG


def _body(x_hbm, wt_hbm, pos_hbm, out_hbm, idx_v, rows_v, pos_v, gsem):
    wid = lax.axis_index("s") * NC + lax.axis_index("c")
    pltpu.sync_copy(pos_hbm, pos_v)

    def chunk_body(c, carry):
        base = wid * ROWS_PER_W + c * CHUNK
        pltpu.sync_copy(x_hbm.at[pl.ds(base, CHUNK)], idx_v)
        cps = [
            pltpu.async_copy(
                wt_hbm.at[idx_v.at[pl.ds(k * SUBG, SUBG)]],
                rows_v.at[pl.ds(k * SUBG, SUBG)],
                gsem,
            )
            for k in range(NSUB)
        ]
        for cp in cps:
            cp.wait()

        def add_body(p, carry2):
            for j in range(EMB_DIM // 16):
                pv = pos_v[p, pl.ds(j * 16, 16)]
                for s in range(SEQ_PER_CHUNK):
                    plsc.addupdate(rows_v.at[s * SEQ_LEN + p, pl.ds(j * 16, 16)], pv)
            return carry2

        lax.fori_loop(0, SEQ_LEN, add_body, 0)
        pltpu.sync_copy(rows_v, out_hbm.at[pl.ds(base, CHUNK)])
        return carry

    lax.fori_loop(0, N_CHUNKS, chunk_body, 0)


@jax.jit
def kernel(x, word_table, pos_table):
    x_flat = x.reshape(TOTAL_ROWS)
    mesh = plsc.VectorSubcoreMesh(core_axis_name="c", subcore_axis_name="s")
    out = pl.kernel(
        _body,
        out_type=jax.ShapeDtypeStruct((TOTAL_ROWS, EMB_DIM), jnp.float32),
        mesh=mesh,
        scratch_types=[
            pltpu.VMEM((CHUNK,), jnp.int32),
            pltpu.VMEM((CHUNK, EMB_DIM), jnp.float32),
            pltpu.VMEM((MAX_LEN, EMB_DIM), jnp.float32),
            pltpu.SemaphoreType.DMA,
        ],
        compiler_params=pltpu.CompilerParams(use_tc_tiling_on_sc=False),
    )(x_flat, word_table, pos_table)
    return out.reshape(BATCH, SEQ_LEN, EMB_DIM)


# speedup vs baseline: 4.1745x; 1.2081x over previous
import jax
import jax.numpy as jnp
from jax import lax
from jax.experimental import pallas as pl
from jax.experimental.pallas import tpu as pltpu
from jax.experimental.pallas import tpu_sc as plsc

VOCAB = 100000
MAX_LEN = 200
EMB_DIM = 64
BATCH = 4096
SEQ_LEN = 200

NC, NS = 2, 16
NW = NC * NS
TOTAL_ROWS = BATCH * SEQ_LEN
ROWS_PER_W = TOTAL_ROWS // NW
SEQ_PER_CHUNK = 2
CHUNK = SEQ_PER_CHUNK * SEQ_LEN
N_CHUNKS = ROWS_PER_W // CHUNK
SUBG = 80
NSUB = CHUNK // SUBG


def _body(x_hbm, wt_hbm, pos_hbm, out_hbm,
          idx0, idx1, rows0, rows1, pos_v,
          isem0, isem1, gsem0, gsem1, osem0, osem1):
    idx = [idx0, idx1]
    rows = [rows0, rows1]
    isem = [isem0, isem1]
    gsem = [gsem0, gsem1]
    osem = [osem0, osem1]

    wid = lax.axis_index("s") * NC + lax.axis_index("c")
    w_base = wid * ROWS_PER_W

    def start_gather(g, b):
        for k in range(NSUB):
            pltpu.async_copy(
                wt_hbm.at[idx[b].at[pl.ds(k * SUBG, SUBG)]],
                rows[b].at[pl.ds(k * SUBG, SUBG)],
                gsem[b],
            )

    def wait_gather(b):
        pltpu.make_async_copy(out_hbm.at[pl.ds(0, CHUNK)], rows[b], gsem[b]).wait()

    def wait_out(b):
        pltpu.make_async_copy(rows[b], out_hbm.at[pl.ds(0, CHUNK)], osem[b]).wait()

    def add_pos(b):
        def add_body(p, carry):
            for j in range(EMB_DIM // 16):
                pv = pos_v[p, pl.ds(j * 16, 16)]
                for s in range(SEQ_PER_CHUNK):
                    plsc.addupdate(rows[b].at[s * SEQ_LEN + p, pl.ds(j * 16, 16)], pv)
            return carry
        lax.fori_loop(0, SEQ_LEN, add_body, 0)

    pltpu.sync_copy(pos_hbm, pos_v)
    pltpu.sync_copy(x_hbm.at[pl.ds(w_base, CHUNK)], idx[0])
    start_gather(0, 0)
    pltpu.async_copy(x_hbm.at[pl.ds(w_base + CHUNK, CHUNK)], idx[1], isem[1])

    def half(g, a):
        b = 1 - a

        @pl.when(g > 0)
        def _():
            @pl.when(g + 1 < N_CHUNKS)
            def _():
                wait_out(b)

        @pl.when(g + 1 < N_CHUNKS)
        def _():
            pltpu.make_async_copy(
                x_hbm.at[pl.ds(0, CHUNK)], idx[b], isem[b]).wait()
            start_gather(g + 1, b)

        wait_gather(a)

        @pl.when(g + 2 < N_CHUNKS)
        def _():
            pltpu.async_copy(
                x_hbm.at[pl.ds(w_base + (g + 2) * CHUNK, CHUNK)],
                idx[a], isem[a])

        add_pos(a)
        pltpu.async_copy(rows[a], out_hbm.at[pl.ds(w_base + g * CHUNK, CHUNK)],
                         osem[a])

    def pair_body(t, carry):
        half(2 * t, 0)
        half(2 * t + 1, 1)
        return carry

    lax.fori_loop(0, N_CHUNKS // 2, pair_body, 0)
    wait_out(0)
    wait_out(1)


@jax.jit
def kernel(x, word_table, pos_table):
    x_flat = x.reshape(TOTAL_ROWS)
    mesh = plsc.VectorSubcoreMesh(core_axis_name="c", subcore_axis_name="s")
    out = pl.kernel(
        _body,
        out_type=jax.ShapeDtypeStruct((TOTAL_ROWS, EMB_DIM), jnp.float32),
        mesh=mesh,
        scratch_types=[
            pltpu.VMEM((CHUNK,), jnp.int32),
            pltpu.VMEM((CHUNK,), jnp.int32),
            pltpu.VMEM((CHUNK, EMB_DIM), jnp.float32),
            pltpu.VMEM((CHUNK, EMB_DIM), jnp.float32),
            pltpu.VMEM((MAX_LEN, EMB_DIM), jnp.float32),
            pltpu.SemaphoreType.DMA,
            pltpu.SemaphoreType.DMA,
            pltpu.SemaphoreType.DMA,
            pltpu.SemaphoreType.DMA,
            pltpu.SemaphoreType.DMA,
            pltpu.SemaphoreType.DMA,
        ],
        compiler_params=pltpu.CompilerParams(use_tc_tiling_on_sc=False),
    )(x_flat, word_table, pos_table)
    return out.reshape(BATCH, SEQ_LEN, EMB_DIM)
